# Initial kernel scaffold; baseline (speedup 1.0000x reference)
import jax
import jax.numpy as jnp
from jax.experimental import pallas as pl


def kernel(prediction, targets, head_size):
    raise NotImplementedError("write your pallas kernel here")


# speedup vs baseline: 165.6167x; 165.6167x over previous
import math

import jax
import jax.numpy as jnp
from jax.experimental import pallas as pl
from jax.experimental.pallas import tpu as pltpu

_KS = 31
_HALF = (_KS - 1) // 2
_SIGMA = 2.0
_H = 256
_W = 256
_S = sum(math.exp(-((i - _HALF) ** 2) / (2.0 * _SIGMA * _SIGMA)) for i in range(_KS))
_INV_S = 1.0 / _S
_NEG_HALF_INV_VAR = -1.0 / (2.0 * _SIGMA * _SIGMA)


def _plane_kernel(tgt_ref, pred_ref, thm_ref, stats_ref):
    p = pl.program_id(0)
    tx = tgt_ref[p, 0]
    ty = tgt_ref[p, 1]

    xi = jax.lax.broadcasted_iota(jnp.int32, (1, _W), 1).astype(jnp.float32)

    def taps(t):
        t0 = jnp.floor(t)
        r = t - t0
        d = xi - t0
        e1 = jnp.where((d >= -15.0) & (d <= 15.0),
                       jnp.exp(d * d * _NEG_HALF_INV_VAR), 0.0)
        d2 = d - 1.0
        e2 = jnp.where((d2 >= -15.0) & (d2 <= 15.0),
                       jnp.exp(d2 * d2 * _NEG_HALF_INV_VAR), 0.0)
        return ((1.0 - r) * e1 + r * e2) * _INV_S

    vx = taps(tx)
    vy = taps(ty)

    si = jax.lax.broadcasted_iota(jnp.int32, (8, _W), 0)
    vy8 = jnp.where(si == 0, vy, 0.0)
    vx8 = jnp.where(si == 0, vx, 0.0)
    thm = jax.lax.dot_general(vy8, vx8, (((0,), (0,)), ((), ())),
                              preferred_element_type=jnp.float32)

    pred = pred_ref[0]
    diff = thm - pred
    lsum = jnp.sum(jnp.sum(diff * diff, axis=0, keepdims=True),
                   axis=1, keepdims=True)

    m = jnp.max(jnp.max(pred, axis=0, keepdims=True), axis=1, keepdims=True)
    fy = jax.lax.broadcasted_iota(jnp.int32, (_H, _W), 0)
    fx = jax.lax.broadcasted_iota(jnp.int32, (_H, _W), 1)
    flat_idx = (fy * _W + fx).astype(jnp.float32)
    cand = jnp.where(pred == m, flat_idx, 1e9)
    idxf = jnp.min(jnp.min(cand, axis=0, keepdims=True), axis=1, keepdims=True)

    thm_ref[0] = thm
    li = jax.lax.broadcasted_iota(jnp.int32, (1, 1, 128), 2)
    stats_ref[...] = jnp.where(li == 0, lsum.reshape(1, 1, 1), idxf.reshape(1, 1, 1))


def _finish_kernel(idx_ref, lsum_ref, tx_ref, ty_ref, vis_ref, head_ref,
                   loss_ref, mean_err_ref, xp_ref, yp_ref, pckh_ref, err_ref):
    idxf = idx_ref[...]
    y_pred = jnp.floor(idxf * (1.0 / float(_W)))
    x_pred = idxf - y_pred * float(_W)

    dx = x_pred - tx_ref[...]
    dy = y_pred - ty_ref[...]
    err = jnp.sqrt(dx * dx + dy * dy)
    vis = vis_ref[...]
    denom = 0.001 + jnp.sum(vis, keepdims=True)
    mean_err = jnp.sum(err * vis, keepdims=True) / denom

    thr = head_ref[...] * 0.5
    inliers = (err <= thr).astype(jnp.float32)
    pckh = jnp.sum(inliers * vis, keepdims=True) / denom

    n_planes = float(idxf.shape[0] * idxf.shape[1])
    loss = jnp.sum(lsum_ref[...], keepdims=True) * (1.0 / n_planes)

    loss_ref[...] = loss
    mean_err_ref[...] = mean_err
    pckh_ref[...] = pckh
    xp_ref[...] = x_pred
    yp_ref[...] = y_pred
    err_ref[...] = err


def kernel(prediction, targets, head_size):
    B, C, H, W = prediction.shape
    n = B * C
    pred3 = prediction.reshape(n, H, W)
    tflat = targets.reshape(n, 3)

    thm3, stats = pl.pallas_call(
        _plane_kernel,
        grid=(n,),
        in_specs=[
            pl.BlockSpec(memory_space=pltpu.SMEM),
            pl.BlockSpec((1, H, W), lambda p: (p, 0, 0)),
        ],
        out_specs=[
            pl.BlockSpec((1, H, W), lambda p: (p, 0, 0)),
            pl.BlockSpec((1, 1, 128), lambda p: (p, 0, 0)),
        ],
        out_shape=[
            jax.ShapeDtypeStruct((n, H, W), jnp.float32),
            jax.ShapeDtypeStruct((n, 1, 128), jnp.float32),
        ],
        compiler_params=pltpu.CompilerParams(
            dimension_semantics=("parallel",)),
    )(tflat, pred3)

    target_heat_map = thm3.reshape(B, C, H, W)
    lsum = stats[:, 0, 0].reshape(B, C)
    idxf = stats[:, 0, 1].reshape(B, C)

    loss, mean_err, x_pred, y_pred, pckh, err = pl.pallas_call(
        _finish_kernel,
        out_shape=[
            jax.ShapeDtypeStruct((1, 1), jnp.float32),
            jax.ShapeDtypeStruct((1, 1), jnp.float32),
            jax.ShapeDtypeStruct((B, C), jnp.float32),
            jax.ShapeDtypeStruct((B, C), jnp.float32),
            jax.ShapeDtypeStruct((1, 1), jnp.float32),
            jax.ShapeDtypeStruct((B, C), jnp.float32),
        ],
    )(idxf, lsum, targets[..., 0], targets[..., 1], targets[..., 2],
      head_size.reshape(B, 1))

    pred_joints = jnp.stack([x_pred, y_pred], axis=-1)
    return (loss[0, 0], mean_err[0, 0], pred_joints, target_heat_map,
            pckh[0, 0], err)


# speedup vs baseline: 451.1612x; 2.7241x over previous
import math

import jax
import jax.numpy as jnp
from jax.experimental import pallas as pl
from jax.experimental.pallas import tpu as pltpu

_KS = 31
_HALF = (_KS - 1) // 2
_SIGMA = 2.0
_H = 256
_W = 256
_S = sum(math.exp(-((i - _HALF) ** 2) / (2.0 * _SIGMA * _SIGMA)) for i in range(_KS))
_INV_S = 1.0 / _S
_NEG_HALF_INV_VAR = -1.0 / (2.0 * _SIGMA * _SIGMA)


_G = 8


def _plane_kernel(tgt_ref, pred_ref, thm_ref, stats_ref):
    pbase = pl.program_id(0) * _G

    xi = jax.lax.broadcasted_iota(jnp.int32, (1, _W), 1).astype(jnp.float32)
    si = jax.lax.broadcasted_iota(jnp.int32, (8, _W), 0)
    fy = jax.lax.broadcasted_iota(jnp.int32, (_H, _W), 0)
    fx = jax.lax.broadcasted_iota(jnp.int32, (_H, _W), 1)
    flat_idx = (fy * _W + fx).astype(jnp.float32)
    li = jax.lax.broadcasted_iota(jnp.int32, (1, 128), 1)

    def taps(t):
        t0 = jnp.floor(t)
        r = t - t0
        d = xi - t0
        e1 = jnp.where((d >= -15.0) & (d <= 15.0),
                       jnp.exp(d * d * _NEG_HALF_INV_VAR), 0.0)
        d2 = d - 1.0
        e2 = jnp.where((d2 >= -15.0) & (d2 <= 15.0),
                       jnp.exp(d2 * d2 * _NEG_HALF_INV_VAR), 0.0)
        return ((1.0 - r) * e1 + r * e2) * _INV_S

    for g in range(_G):
        tx = tgt_ref[pbase + g, 0]
        ty = tgt_ref[pbase + g, 1]
        vx = taps(tx)
        vy = taps(ty)

        vy8 = jnp.where(si == 0, vy, 0.0)
        vx8 = jnp.where(si == 0, vx, 0.0)
        thm = jax.lax.dot_general(vy8, vx8, (((0,), (0,)), ((), ())),
                                  preferred_element_type=jnp.float32)

        pred = pred_ref[g]
        diff = thm - pred
        lsum = jnp.sum(jnp.sum(diff * diff, axis=0, keepdims=True),
                       axis=1, keepdims=True)

        m = jnp.max(jnp.max(pred, axis=0, keepdims=True),
                    axis=1, keepdims=True)
        cand = jnp.where(pred == m, flat_idx, 1e9)
        idxf = jnp.min(jnp.min(cand, axis=0, keepdims=True),
                       axis=1, keepdims=True)

        thm_ref[g] = thm
        stats_ref[g] = jnp.where(li == 0, lsum, idxf)


def _finish_kernel(idx_ref, lsum_ref, tx_ref, ty_ref, vis_ref, head_ref,
                   loss_ref, mean_err_ref, xp_ref, yp_ref, pckh_ref, err_ref):
    idxf = idx_ref[...]
    y_pred = jnp.floor(idxf * (1.0 / float(_W)))
    x_pred = idxf - y_pred * float(_W)

    dx = x_pred - tx_ref[...]
    dy = y_pred - ty_ref[...]
    err = jnp.sqrt(dx * dx + dy * dy)
    vis = vis_ref[...]
    denom = 0.001 + jnp.sum(vis, keepdims=True)
    mean_err = jnp.sum(err * vis, keepdims=True) / denom

    thr = head_ref[...] * 0.5
    inliers = (err <= thr).astype(jnp.float32)
    pckh = jnp.sum(inliers * vis, keepdims=True) / denom

    n_planes = float(idxf.shape[0] * idxf.shape[1])
    loss = jnp.sum(lsum_ref[...], keepdims=True) * (1.0 / n_planes)

    loss_ref[...] = loss
    mean_err_ref[...] = mean_err
    pckh_ref[...] = pckh
    xp_ref[...] = x_pred
    yp_ref[...] = y_pred
    err_ref[...] = err


def kernel(prediction, targets, head_size):
    B, C, H, W = prediction.shape
    n = B * C
    pred3 = prediction.reshape(n, H, W)
    tflat = targets.reshape(n, 3)

    thm3, stats = pl.pallas_call(
        _plane_kernel,
        grid=(n // _G,),
        in_specs=[
            pl.BlockSpec(memory_space=pltpu.SMEM),
            pl.BlockSpec((_G, H, W), lambda p: (p, 0, 0)),
        ],
        out_specs=[
            pl.BlockSpec((_G, H, W), lambda p: (p, 0, 0)),
            pl.BlockSpec((_G, 1, 128), lambda p: (p, 0, 0)),
        ],
        out_shape=[
            jax.ShapeDtypeStruct((n, H, W), jnp.float32),
            jax.ShapeDtypeStruct((n, 1, 128), jnp.float32),
        ],
        compiler_params=pltpu.CompilerParams(
            dimension_semantics=("parallel",)),
    )(tflat, pred3)

    target_heat_map = thm3.reshape(B, C, H, W)
    lsum = stats[:, 0, 0].reshape(B, C)
    idxf = stats[:, 0, 1].reshape(B, C)

    loss, mean_err, x_pred, y_pred, pckh, err = pl.pallas_call(
        _finish_kernel,
        out_shape=[
            jax.ShapeDtypeStruct((1, 1), jnp.float32),
            jax.ShapeDtypeStruct((1, 1), jnp.float32),
            jax.ShapeDtypeStruct((B, C), jnp.float32),
            jax.ShapeDtypeStruct((B, C), jnp.float32),
            jax.ShapeDtypeStruct((1, 1), jnp.float32),
            jax.ShapeDtypeStruct((B, C), jnp.float32),
        ],
    )(idxf, lsum, targets[..., 0], targets[..., 1], targets[..., 2],
      head_size.reshape(B, 1))

    pred_joints = jnp.stack([x_pred, y_pred], axis=-1)
    return (loss[0, 0], mean_err[0, 0], pred_joints, target_heat_map,
            pckh[0, 0], err)


# speedup vs baseline: 589.9777x; 1.3077x over previous
import math

import jax
import jax.numpy as jnp
from jax.experimental import pallas as pl
from jax.experimental.pallas import tpu as pltpu

_KS = 31
_HALF = (_KS - 1) // 2
_SIGMA = 2.0
_H = 256
_W = 256
_S = sum(math.exp(-((i - _HALF) ** 2) / (2.0 * _SIGMA * _SIGMA)) for i in range(_KS))
_INV_S = 1.0 / _S
_NEG_HALF_INV_VAR = -1.0 / (2.0 * _SIGMA * _SIGMA)

_G = 26
_C = 13


def _plane_kernel(tgt_ref, head_ref, pred_ref, thm_ref, stats_ref, fin_ref,
                  acc_ref):
    p = pl.program_id(0)
    n_steps = pl.num_programs(0)
    pbase = p * _G

    @pl.when(p == 0)
    def _():
        acc_ref[...] = jnp.zeros_like(acc_ref)

    xi = jax.lax.broadcasted_iota(jnp.int32, (1, _W), 1).astype(jnp.float32)
    si = jax.lax.broadcasted_iota(jnp.int32, (8, _W), 0)
    fy = jax.lax.broadcasted_iota(jnp.int32, (_H, _W), 0)
    fx = jax.lax.broadcasted_iota(jnp.int32, (_H, _W), 1)
    flat_idx = (fy * _W + fx).astype(jnp.float32)
    li = jax.lax.broadcasted_iota(jnp.int32, (1, 128), 1)
    si4 = jax.lax.broadcasted_iota(jnp.int32, (4, 128), 0)

    def taps(t):
        t0 = jnp.floor(t)
        r = t - t0
        d = xi - t0
        e1 = jnp.where((d >= -15.0) & (d <= 15.0),
                       jnp.exp(d * d * _NEG_HALF_INV_VAR), 0.0)
        d2 = d - 1.0
        e2 = jnp.where((d2 >= -15.0) & (d2 <= 15.0),
                       jnp.exp(d2 * d2 * _NEG_HALF_INV_VAR), 0.0)
        return ((1.0 - r) * e1 + r * e2) * _INV_S

    acc_step = jnp.zeros((4, 128), jnp.float32)
    for g in range(_G):
        tx = tgt_ref[pbase + g, 0]
        ty = tgt_ref[pbase + g, 1]
        vis = tgt_ref[pbase + g, 2]
        thr = head_ref[(pbase + g) // _C] * 0.5
        vx = taps(tx)
        vy = taps(ty)

        vy8 = jnp.where(si == 0, vy, 0.0)
        vx8 = jnp.where(si == 0, vx, 0.0)
        thm = jax.lax.dot_general(vy8, vx8, (((0,), (0,)), ((), ())),
                                  preferred_element_type=jnp.float32)

        pred = pred_ref[g]
        diff = thm - pred
        lsum = jnp.sum(jnp.sum(diff * diff, axis=0, keepdims=True),
                       axis=1, keepdims=True)

        m = jnp.max(jnp.max(pred, axis=0, keepdims=True),
                    axis=1, keepdims=True)
        cand = jnp.where(pred == m, flat_idx, 1e9)
        idxf = jnp.min(jnp.min(cand, axis=0, keepdims=True),
                       axis=1, keepdims=True)

        y_pred = jnp.floor(idxf * (1.0 / float(_W)))
        x_pred = idxf - y_pred * float(_W)
        dx = x_pred - tx
        dy = y_pred - ty
        err = jnp.sqrt(dx * dx + dy * dy)
        inl = jnp.where(err <= thr, 1.0, 0.0)

        thm_ref[g] = thm
        stats_ref[g] = jnp.where(li == 0, x_pred,
                                 jnp.where(li == 1, y_pred,
                                           jnp.where(li == 2, err, 0.0)))
        acc_step = acc_step + jnp.where(
            si4 == 0, lsum,
            jnp.where(si4 == 1, vis,
                      jnp.where(si4 == 2, err * vis, inl * vis)))

    acc_ref[...] = acc_ref[...] + acc_step

    @pl.when(p == n_steps - 1)
    def _():
        a = acc_ref[...]
        n_planes = float(_G * n_steps)
        loss = a[0:1, :] / n_planes
        denom = 0.001 + a[1:2, :]
        mean_err = a[2:3, :] / denom
        pckh = a[3:4, :] / denom
        fin_ref[0] = jnp.where(li == 0, loss,
                               jnp.where(li == 1, mean_err,
                                         jnp.where(li == 2, pckh, 0.0)))


def kernel(prediction, targets, head_size):
    B, C, H, W = prediction.shape
    n = B * C
    pred3 = prediction.reshape(n, H, W)
    tflat = targets.reshape(n, 3)

    thm3, stats, fin = pl.pallas_call(
        _plane_kernel,
        grid=(n // _G,),
        in_specs=[
            pl.BlockSpec(memory_space=pltpu.SMEM),
            pl.BlockSpec(memory_space=pltpu.SMEM),
            pl.BlockSpec((_G, H, W), lambda p: (p, 0, 0)),
        ],
        out_specs=[
            pl.BlockSpec((_G, H, W), lambda p: (p, 0, 0)),
            pl.BlockSpec((_G, 1, 128), lambda p: (p, 0, 0)),
            pl.BlockSpec((1, 1, 128), lambda p: (0, 0, 0)),
        ],
        out_shape=[
            jax.ShapeDtypeStruct((n, H, W), jnp.float32),
            jax.ShapeDtypeStruct((n, 1, 128), jnp.float32),
            jax.ShapeDtypeStruct((1, 1, 128), jnp.float32),
        ],
        scratch_shapes=[pltpu.VMEM((4, 128), jnp.float32)],
        compiler_params=pltpu.CompilerParams(
            dimension_semantics=("arbitrary",)),
    )(tflat, head_size, pred3)

    target_heat_map = thm3.reshape(B, C, H, W)
    pred_joints = stats[:, 0, 0:2].reshape(B, C, 2)
    position_error_2d = stats[:, 0, 2].reshape(B, C)
    return (fin[0, 0, 0], fin[0, 0, 1], pred_joints, target_heat_map,
            fin[0, 0, 2], position_error_2d)


# speedup vs baseline: 595.5396x; 1.0094x over previous
import math

import jax
import jax.numpy as jnp
from jax.experimental import pallas as pl
from jax.experimental.pallas import tpu as pltpu

_KS = 31
_HALF = (_KS - 1) // 2
_SIGMA = 2.0
_H = 256
_W = 256
_S = sum(math.exp(-((i - _HALF) ** 2) / (2.0 * _SIGMA * _SIGMA)) for i in range(_KS))
_INV_S = 1.0 / _S
_NEG_HALF_INV_VAR = -1.0 / (2.0 * _SIGMA * _SIGMA)

_G = 16
_HALVES = 2
_C = 13


def _plane_kernel(tgt_ref, head_ref, pred_a_ref, pred_b_ref, thm_ref,
                  stats_ref, fin_ref, acc_ref):
    p = pl.program_id(0)
    n_steps = pl.num_programs(0)
    pbase = p * (_HALVES * _G)

    @pl.when(p == 0)
    def _():
        acc_ref[...] = jnp.zeros_like(acc_ref)

    xi = jax.lax.broadcasted_iota(jnp.int32, (1, _W), 1).astype(jnp.float32)
    si = jax.lax.broadcasted_iota(jnp.int32, (8, _W), 0)
    fy = jax.lax.broadcasted_iota(jnp.int32, (_H, _W), 0)
    fx = jax.lax.broadcasted_iota(jnp.int32, (_H, _W), 1)
    flat_idx = (fy * _W + fx).astype(jnp.float32)
    li = jax.lax.broadcasted_iota(jnp.int32, (1, 128), 1)
    si4 = jax.lax.broadcasted_iota(jnp.int32, (4, 128), 0)

    def taps(t):
        t0 = jnp.floor(t)
        r = t - t0
        d = xi - t0
        e1 = jnp.where((d >= -15.0) & (d <= 15.0),
                       jnp.exp(d * d * _NEG_HALF_INV_VAR), 0.0)
        d2 = d - 1.0
        e2 = jnp.where((d2 >= -15.0) & (d2 <= 15.0),
                       jnp.exp(d2 * d2 * _NEG_HALF_INV_VAR), 0.0)
        return ((1.0 - r) * e1 + r * e2) * _INV_S

    acc_step = jnp.zeros((4, 128), jnp.float32)
    for g in range(_HALVES * _G):
        tx = tgt_ref[pbase + g, 0]
        ty = tgt_ref[pbase + g, 1]
        vis = tgt_ref[pbase + g, 2]
        thr = head_ref[(pbase + g) // _C] * 0.5
        vx = taps(tx)
        vy = taps(ty)

        vy8 = jnp.where(si == 0, vy, 0.0)
        vx8 = jnp.where(si == 0, vx, 0.0)
        thm = jax.lax.dot_general(vy8, vx8, (((0,), (0,)), ((), ())),
                                  preferred_element_type=jnp.float32)

        pred = pred_a_ref[g] if g < _G else pred_b_ref[g - _G]
        diff = thm - pred
        lsum = jnp.sum(jnp.sum(diff * diff, axis=0, keepdims=True),
                       axis=1, keepdims=True)

        m = jnp.max(jnp.max(pred, axis=0, keepdims=True),
                    axis=1, keepdims=True)
        cand = jnp.where(pred == m, flat_idx, 1e9)
        idxf = jnp.min(jnp.min(cand, axis=0, keepdims=True),
                       axis=1, keepdims=True)

        y_pred = jnp.floor(idxf * (1.0 / float(_W)))
        x_pred = idxf - y_pred * float(_W)
        dx = x_pred - tx
        dy = y_pred - ty
        err = jnp.sqrt(dx * dx + dy * dy)
        inl = jnp.where(err <= thr, 1.0, 0.0)

        thm_ref[g] = thm
        stats_ref[g] = jnp.where(li == 0, x_pred,
                                 jnp.where(li == 1, y_pred,
                                           jnp.where(li == 2, err, 0.0)))
        acc_step = acc_step + jnp.where(
            si4 == 0, lsum,
            jnp.where(si4 == 1, vis,
                      jnp.where(si4 == 2, err * vis, inl * vis)))

    acc_ref[...] = acc_ref[...] + acc_step

    @pl.when(p == n_steps - 1)
    def _():
        a = acc_ref[...]
        n_planes = float(_HALVES * _G * n_steps)
        loss = a[0:1, :] / n_planes
        denom = 0.001 + a[1:2, :]
        mean_err = a[2:3, :] / denom
        pckh = a[3:4, :] / denom
        fin_ref[0] = jnp.where(li == 0, loss,
                               jnp.where(li == 1, mean_err,
                                         jnp.where(li == 2, pckh, 0.0)))


def kernel(prediction, targets, head_size):
    B, C, H, W = prediction.shape
    n = B * C
    pred3 = prediction.reshape(n, H, W)
    tflat = targets.reshape(n, 3)

    step = _HALVES * _G
    thm3, stats, fin = pl.pallas_call(
        _plane_kernel,
        grid=(n // step,),
        in_specs=[
            pl.BlockSpec(memory_space=pltpu.SMEM),
            pl.BlockSpec(memory_space=pltpu.SMEM),
            pl.BlockSpec((_G, H, W), lambda p: (2 * p, 0, 0)),
            pl.BlockSpec((_G, H, W), lambda p: (2 * p + 1, 0, 0)),
        ],
        out_specs=[
            pl.BlockSpec((step, H, W), lambda p: (p, 0, 0)),
            pl.BlockSpec((step, 1, 128), lambda p: (p, 0, 0)),
            pl.BlockSpec((1, 1, 128), lambda p: (0, 0, 0)),
        ],
        out_shape=[
            jax.ShapeDtypeStruct((n, H, W), jnp.float32),
            jax.ShapeDtypeStruct((n, 1, 128), jnp.float32),
            jax.ShapeDtypeStruct((1, 1, 128), jnp.float32),
        ],
        scratch_shapes=[pltpu.VMEM((4, 128), jnp.float32)],
        compiler_params=pltpu.CompilerParams(
            dimension_semantics=("arbitrary",),
            vmem_limit_bytes=64 * 1024 * 1024),
    )(tflat, head_size, pred3, pred3)

    target_heat_map = thm3.reshape(B, C, H, W)
    pred_joints = stats[:, 0, 0:2].reshape(B, C, 2)
    position_error_2d = stats[:, 0, 2].reshape(B, C)
    return (fin[0, 0, 0], fin[0, 0, 1], pred_joints, target_heat_map,
            fin[0, 0, 2], position_error_2d)


# speedup vs baseline: 611.9094x; 1.0275x over previous
import math

import jax
import jax.numpy as jnp
from jax.experimental import pallas as pl
from jax.experimental.pallas import tpu as pltpu

_KS = 31
_HALF = (_KS - 1) // 2
_SIGMA = 2.0
_H = 256
_W = 256
_S = sum(math.exp(-((i - _HALF) ** 2) / (2.0 * _SIGMA * _SIGMA)) for i in range(_KS))
_INV_S = 1.0 / _S
_NEG_HALF_INV_VAR = -1.0 / (2.0 * _SIGMA * _SIGMA)

_G = 16
_HALVES = 2
_C = 13


def _plane_kernel(tgt_ref, head_ref, pred_a_ref, pred_b_ref, thm_ref,
                  stats_ref, fin_ref, acc_ref):
    p = pl.program_id(0)
    n_steps = pl.num_programs(0)
    pbase = p * (_HALVES * _G)

    @pl.when(p == 0)
    def _():
        acc_ref[...] = jnp.zeros_like(acc_ref)

    xi = jax.lax.broadcasted_iota(jnp.int32, (1, _W), 1).astype(jnp.float32)
    si = jax.lax.broadcasted_iota(jnp.int32, (8, _W), 0)
    li = jax.lax.broadcasted_iota(jnp.int32, (1, 128), 1)
    si4 = jax.lax.broadcasted_iota(jnp.int32, (4, 128), 0)

    def taps(t):
        t0 = jnp.floor(t)
        r = t - t0
        d = xi - t0
        e1 = jnp.where((d >= -15.0) & (d <= 15.0),
                       jnp.exp(d * d * _NEG_HALF_INV_VAR), 0.0)
        d2 = d - 1.0
        e2 = jnp.where((d2 >= -15.0) & (d2 <= 15.0),
                       jnp.exp(d2 * d2 * _NEG_HALF_INV_VAR), 0.0)
        return ((1.0 - r) * e1 + r * e2) * _INV_S

    acc_step = jnp.zeros((4, 128), jnp.float32)
    for g in range(_HALVES * _G):
        tx = tgt_ref[pbase + g, 0]
        ty = tgt_ref[pbase + g, 1]
        vis = tgt_ref[pbase + g, 2]
        thr = head_ref[(pbase + g) // _C] * 0.5
        vx = taps(tx)
        vy = taps(ty)

        vy8 = jnp.where(si == 0, vy, 0.0)
        vx8 = jnp.where(si == 0, vx, 0.0)
        thm = jax.lax.dot_general(vy8, vx8, (((0,), (0,)), ((), ())),
                                  preferred_element_type=jnp.float32)

        pred = pred_a_ref[g] if g < _G else pred_b_ref[g - _G]
        diff = thm - pred
        lsum = jnp.sum(jnp.sum(diff * diff, axis=0, keepdims=True),
                       axis=1, keepdims=True)

        my = jnp.max(pred, axis=0, keepdims=True)
        ay = jnp.argmax(pred, axis=0, keepdims=True)
        m = jnp.max(my, axis=1, keepdims=True)
        fi_row = ay.astype(jnp.float32) * float(_W) + xi
        cand = jnp.where(my == m, fi_row, 1e9)
        idxf = jnp.min(cand, axis=1, keepdims=True)

        y_pred = jnp.floor(idxf * (1.0 / float(_W)))
        x_pred = idxf - y_pred * float(_W)
        dx = x_pred - tx
        dy = y_pred - ty
        err = jnp.sqrt(dx * dx + dy * dy)
        inl = jnp.where(err <= thr, 1.0, 0.0)

        thm_ref[g] = thm
        stats_ref[g] = jnp.where(li == 0, x_pred,
                                 jnp.where(li == 1, y_pred,
                                           jnp.where(li == 2, err, 0.0)))
        acc_step = acc_step + jnp.where(
            si4 == 0, lsum,
            jnp.where(si4 == 1, vis,
                      jnp.where(si4 == 2, err * vis, inl * vis)))

    acc_ref[...] = acc_ref[...] + acc_step

    @pl.when(p == n_steps - 1)
    def _():
        a = acc_ref[...]
        n_planes = float(_HALVES * _G * n_steps)
        loss = a[0:1, :] / n_planes
        denom = 0.001 + a[1:2, :]
        mean_err = a[2:3, :] / denom
        pckh = a[3:4, :] / denom
        fin_ref[0] = jnp.where(li == 0, loss,
                               jnp.where(li == 1, mean_err,
                                         jnp.where(li == 2, pckh, 0.0)))


def kernel(prediction, targets, head_size):
    B, C, H, W = prediction.shape
    n = B * C
    pred3 = prediction.reshape(n, H, W)
    tflat = targets.reshape(n, 3)

    step = _HALVES * _G
    thm3, stats, fin = pl.pallas_call(
        _plane_kernel,
        grid=(n // step,),
        in_specs=[
            pl.BlockSpec(memory_space=pltpu.SMEM),
            pl.BlockSpec(memory_space=pltpu.SMEM),
            pl.BlockSpec((_G, H, W), lambda p: (2 * p, 0, 0)),
            pl.BlockSpec((_G, H, W), lambda p: (2 * p + 1, 0, 0)),
        ],
        out_specs=[
            pl.BlockSpec((step, H, W), lambda p: (p, 0, 0)),
            pl.BlockSpec((step, 1, 128), lambda p: (p, 0, 0)),
            pl.BlockSpec((1, 1, 128), lambda p: (0, 0, 0)),
        ],
        out_shape=[
            jax.ShapeDtypeStruct((n, H, W), jnp.float32),
            jax.ShapeDtypeStruct((n, 1, 128), jnp.float32),
            jax.ShapeDtypeStruct((1, 1, 128), jnp.float32),
        ],
        scratch_shapes=[pltpu.VMEM((4, 128), jnp.float32)],
        compiler_params=pltpu.CompilerParams(
            dimension_semantics=("arbitrary",),
            vmem_limit_bytes=64 * 1024 * 1024),
    )(tflat, head_size, pred3, pred3)

    target_heat_map = thm3.reshape(B, C, H, W)
    pred_joints = stats[:, 0, 0:2].reshape(B, C, 2)
    position_error_2d = stats[:, 0, 2].reshape(B, C)
    return (fin[0, 0, 0], fin[0, 0, 1], pred_joints, target_heat_map,
            fin[0, 0, 2], position_error_2d)


# speedup vs baseline: 618.5926x; 1.0109x over previous
import math

import jax
import jax.numpy as jnp
from jax.experimental import pallas as pl
from jax.experimental.pallas import tpu as pltpu

_KS = 31
_HALF = (_KS - 1) // 2
_SIGMA = 2.0
_H = 256
_W = 256
_S = sum(math.exp(-((i - _HALF) ** 2) / (2.0 * _SIGMA * _SIGMA)) for i in range(_KS))
_INV_S = 1.0 / _S
_NEG_HALF_INV_VAR = -1.0 / (2.0 * _SIGMA * _SIGMA)

_G = 26
_HALVES = 2
_C = 13


def _plane_kernel(tgt_ref, head_ref, pred_a_ref, pred_b_ref, thm_ref,
                  stats_ref, fin_ref, acc_ref):
    p = pl.program_id(0)
    n_steps = pl.num_programs(0)
    pbase = p * (_HALVES * _G)

    @pl.when(p == 0)
    def _():
        acc_ref[...] = jnp.zeros_like(acc_ref)

    xi = jax.lax.broadcasted_iota(jnp.int32, (1, _W), 1).astype(jnp.float32)
    si = jax.lax.broadcasted_iota(jnp.int32, (8, _W), 0)
    li = jax.lax.broadcasted_iota(jnp.int32, (1, 128), 1)
    si4 = jax.lax.broadcasted_iota(jnp.int32, (4, 128), 0)

    def taps(t):
        t0 = jnp.floor(t)
        r = t - t0
        d = xi - t0
        e1 = jnp.where((d >= -15.0) & (d <= 15.0),
                       jnp.exp(d * d * _NEG_HALF_INV_VAR), 0.0)
        d2 = d - 1.0
        e2 = jnp.where((d2 >= -15.0) & (d2 <= 15.0),
                       jnp.exp(d2 * d2 * _NEG_HALF_INV_VAR), 0.0)
        return ((1.0 - r) * e1 + r * e2) * _INV_S

    acc_step = jnp.zeros((4, 128), jnp.float32)
    for g in range(_HALVES * _G):
        tx = tgt_ref[pbase + g, 0]
        ty = tgt_ref[pbase + g, 1]
        vis = tgt_ref[pbase + g, 2]
        thr = head_ref[(pbase + g) // _C] * 0.5
        vx = taps(tx)
        vy = taps(ty)

        vy8 = jnp.where(si == 0, vy, 0.0)
        vx8 = jnp.where(si == 0, vx, 0.0)
        thm = jax.lax.dot_general(vy8, vx8, (((0,), (0,)), ((), ())),
                                  preferred_element_type=jnp.float32)

        pred = pred_a_ref[g] if g < _G else pred_b_ref[g - _G]
        diff = thm - pred
        lsum = jnp.sum(jnp.sum(diff * diff, axis=0, keepdims=True),
                       axis=1, keepdims=True)

        my = jnp.max(pred, axis=0, keepdims=True)
        ay = jnp.argmax(pred, axis=0, keepdims=True)
        m = jnp.max(my, axis=1, keepdims=True)
        fi_row = ay.astype(jnp.float32) * float(_W) + xi
        cand = jnp.where(my == m, fi_row, 1e9)
        idxf = jnp.min(cand, axis=1, keepdims=True)

        y_pred = jnp.floor(idxf * (1.0 / float(_W)))
        x_pred = idxf - y_pred * float(_W)
        dx = x_pred - tx
        dy = y_pred - ty
        err = jnp.sqrt(dx * dx + dy * dy)
        inl = jnp.where(err <= thr, 1.0, 0.0)

        thm_ref[g] = thm
        stats_ref[g] = jnp.where(li == 0, x_pred,
                                 jnp.where(li == 1, y_pred,
                                           jnp.where(li == 2, err, 0.0)))
        acc_step = acc_step + jnp.where(
            si4 == 0, lsum,
            jnp.where(si4 == 1, vis,
                      jnp.where(si4 == 2, err * vis, inl * vis)))

    acc_ref[...] = acc_ref[...] + acc_step

    @pl.when(p == n_steps - 1)
    def _():
        a = acc_ref[...]
        n_planes = float(_HALVES * _G * n_steps)
        loss = a[0:1, :] / n_planes
        denom = 0.001 + a[1:2, :]
        mean_err = a[2:3, :] / denom
        pckh = a[3:4, :] / denom
        fin_ref[0] = jnp.where(li == 0, loss,
                               jnp.where(li == 1, mean_err,
                                         jnp.where(li == 2, pckh, 0.0)))


def kernel(prediction, targets, head_size):
    B, C, H, W = prediction.shape
    n = B * C
    pred3 = prediction.reshape(n, H, W)
    tflat = targets.reshape(n, 3)

    step = _HALVES * _G
    thm3, stats, fin = pl.pallas_call(
        _plane_kernel,
        grid=(n // step,),
        in_specs=[
            pl.BlockSpec(memory_space=pltpu.SMEM),
            pl.BlockSpec(memory_space=pltpu.SMEM),
            pl.BlockSpec((_G, H, W), lambda p: (2 * p, 0, 0)),
            pl.BlockSpec((_G, H, W), lambda p: (2 * p + 1, 0, 0)),
        ],
        out_specs=[
            pl.BlockSpec((step, H, W), lambda p: (p, 0, 0)),
            pl.BlockSpec((step, 1, 128), lambda p: (p, 0, 0)),
            pl.BlockSpec((1, 1, 128), lambda p: (0, 0, 0)),
        ],
        out_shape=[
            jax.ShapeDtypeStruct((n, H, W), jnp.float32),
            jax.ShapeDtypeStruct((n, 1, 128), jnp.float32),
            jax.ShapeDtypeStruct((1, 1, 128), jnp.float32),
        ],
        scratch_shapes=[pltpu.VMEM((4, 128), jnp.float32)],
        compiler_params=pltpu.CompilerParams(
            dimension_semantics=("arbitrary",),
            vmem_limit_bytes=64 * 1024 * 1024),
    )(tflat, head_size, pred3, pred3)

    target_heat_map = thm3.reshape(B, C, H, W)
    pred_joints = stats[:, 0, 0:2].reshape(B, C, 2)
    position_error_2d = stats[:, 0, 2].reshape(B, C)
    return (fin[0, 0, 0], fin[0, 0, 1], pred_joints, target_heat_map,
            fin[0, 0, 2], position_error_2d)
